# Initial kernel scaffold; baseline (speedup 1.0000x reference)
import jax
import jax.numpy as jnp
from jax.experimental import pallas as pl


def kernel(batchVector, nodeEmbedding, relaEmbedding, linkEmbedding, weEmbedding, node_c0, we_c0, timesx_c0, node_c1, we_c1, timesx_c1):
    raise NotImplementedError("write your pallas kernel here")


# speedup vs baseline: 1.4922x; 1.4922x over previous
import functools

import jax
import jax.numpy as jnp
from jax import lax
from jax.experimental import pallas as pl
from jax.experimental.pallas import tpu as pltpu
from jax.experimental.pallas import tpu_sc as plsc

DIM = 128
BATCH = 16384
NC_ROWS = 512
NUM_CORES = 2
NUM_SUBCORES = 16
NW = NUM_CORES * NUM_SUBCORES
RPW = BATCH // NW
CHUNK = 64
NCHUNK = RPW // CHUNK
SEG = DIM // 16
CL_PER_W = NC_ROWS // NW


def _sc_body(idxk, i1, i2, i3, i4, node, rela, link, wepad,
             nc0, wc0, tx0, nc1, wc1, tx1,
             scores, pc0, pc1,
             idx_v, buf, scores_v, crows, we_v, cwe_v, ctx_v, cidx_v,
             pc_v, semA, semB, semC):
  wid = lax.axis_index("s") * NUM_CORES + lax.axis_index("c")
  base = wid * RPW

  def fire(c, s, sem):
    cb = base + c * CHUNK
    for col, src in enumerate((idxk, i1, i2, i3, i4)):
      pltpu.sync_copy(src.at[pl.ds(cb, CHUNK)], idx_v.at[s, col])
    handles = [
        pltpu.async_copy(node.at[idx_v.at[s, 1]], buf.at[s, 0], sem),
        pltpu.async_copy(node.at[idx_v.at[s, 2]], buf.at[s, 1], sem),
        pltpu.async_copy(node.at[idx_v.at[s, 3]], buf.at[s, 2], sem),
        pltpu.async_copy(node.at[idx_v.at[s, 4]], buf.at[s, 3], sem),
        pltpu.async_copy(rela.at[idx_v.at[s, 0]], buf.at[s, 4], sem),
        pltpu.async_copy(link.at[idx_v.at[s, 0]], buf.at[s, 5], sem),
    ]
    return handles

  def compute(c, s):
    def row_body(r, carry):
      acc = jnp.zeros((16,), jnp.float32)
      for g in range(SEG):
        d = pl.ds(g * 16, 16)
        pi = buf[s, 0, r, d]
        pj = buf[s, 1, r, d]
        ni = buf[s, 2, r, d]
        nj = buf[s, 3, r, d]
        rk = buf[s, 4, r, d]
        lk = buf[s, 5, r, d]
        acc = acc + (nj - ni + pi - pj) * rk + (ni * nj - pi * pj) * lk
      scores_v[r, :] = acc
      return carry
    lax.fori_loop(0, CHUNK, row_body, 0)
    pltpu.sync_copy(scores_v, scores.at[pl.ds(base + c * CHUNK, CHUNK)])

  def cluster(nc, wc, tx, pc_out):
    cb = wid * CL_PER_W
    pltpu.sync_copy(nc.at[pl.ds(cb, CL_PER_W)], cidx_v)
    h = pltpu.async_copy(node.at[cidx_v], crows, semC)
    pltpu.sync_copy(wc.at[pl.ds(cb, CL_PER_W)], cwe_v)
    pltpu.sync_copy(tx.at[pl.ds(cb, CL_PER_W)], ctx_v)
    pltpu.sync_copy(wepad, we_v)
    h.wait()
    we_vec = we_v[...]
    cwe_vec = cwe_v[...]
    wex = jnp.zeros((16,), jnp.float32)
    for t in range(8):
      wex = jnp.where(cwe_vec == t, we_vec[t], wex)
    coeff_vec = ctx_v[...] * wex
    crs = [coeff_vec[r] for r in range(CL_PER_W)]
    for g in range(SEG):
      d = pl.ds(g * 16, 16)
      acc = jnp.zeros((16,), jnp.float32)
      for r in range(CL_PER_W):
        acc = acc + crs[r] * crows[r, d]
      pc_v[d] = acc
    pltpu.sync_copy(pc_v, pc_out.at[wid])

  sems = (semA, semB)
  pending = fire(0, 0, sems[0])
  cluster(nc0, wc0, tx0, pc0)
  cluster(nc1, wc1, tx1, pc1)
  for c in range(NCHUNK):
    s = c % 2
    nxt = None
    if c + 1 < NCHUNK:
      nxt = fire(c + 1, 1 - s, sems[1 - s])
    for h in pending:
      h.wait()
    compute(c, s)
    pending = nxt


@functools.partial(
    pl.kernel,
    out_type=(
        jax.ShapeDtypeStruct((BATCH, 16), jnp.float32),
        jax.ShapeDtypeStruct((NW, DIM), jnp.float32),
        jax.ShapeDtypeStruct((NW, DIM), jnp.float32),
    ),
    mesh=plsc.VectorSubcoreMesh(core_axis_name="c", subcore_axis_name="s"),
    scratch_types=[
        pltpu.VMEM((2, 5, CHUNK), jnp.int32),
        pltpu.VMEM((2, 6, CHUNK, DIM), jnp.float32),
        pltpu.VMEM((CHUNK, 16), jnp.float32),
        pltpu.VMEM((CL_PER_W, DIM), jnp.float32),
        pltpu.VMEM((16,), jnp.float32),
        pltpu.VMEM((16,), jnp.int32),
        pltpu.VMEM((16,), jnp.float32),
        pltpu.VMEM((16,), jnp.int32),
        pltpu.VMEM((DIM,), jnp.float32),
        pltpu.SemaphoreType.DMA,
        pltpu.SemaphoreType.DMA,
        pltpu.SemaphoreType.DMA,
    ],
)
def _sc_kernel(*refs):
  _sc_body(*refs)


def _tc_body(scores_ref, pc0_ref, pc1_ref, out_ref):
  s = scores_ref[...]
  err = jnp.sum(s, axis=1, keepdims=True)
  sp = jnp.maximum(err, 0.0) + jnp.log1p(jnp.exp(-jnp.abs(err)))
  loss = jnp.sum(sp)
  c0 = jnp.sum(pc0_ref[...], axis=0) * (1.0 / (NC_ROWS * 1000.0))
  c1 = jnp.sum(pc1_ref[...], axis=0) * (1.0 / (NC_ROWS * 1500.0))
  dvec = c0 - c1
  dis = 0.5 * jnp.sqrt(jnp.sum(dvec * dvec))
  out_ref[0, 0] = loss - 2.0 * dis


_tc_kernel = pl.pallas_call(
    _tc_body,
    out_shape=jax.ShapeDtypeStruct((1, 1), jnp.float32),
    out_specs=pl.BlockSpec(memory_space=pltpu.SMEM),
)


def kernel(batchVector, nodeEmbedding, relaEmbedding, linkEmbedding, weEmbedding,
           node_c0, we_c0, timesx_c0, node_c1, we_c1, timesx_c1):
  cols = batchVector.T
  wepad = jnp.concatenate(
      [weEmbedding.reshape(-1), jnp.zeros((16 - weEmbedding.shape[0],), jnp.float32)])
  scores, pc0, pc1 = _sc_kernel(
      cols[0], cols[1], cols[2], cols[3], cols[4],
      nodeEmbedding, relaEmbedding, linkEmbedding, wepad,
      node_c0, we_c0, timesx_c0.reshape(-1),
      node_c1, we_c1, timesx_c1.reshape(-1))
  total = _tc_kernel(scores, pc0, pc1)
  return total[0, 0]


# speedup vs baseline: 1.8963x; 1.2708x over previous
import functools

import jax
import jax.numpy as jnp
from jax import lax
from jax.experimental import pallas as pl
from jax.experimental.pallas import tpu as pltpu
from jax.experimental.pallas import tpu_sc as plsc

DIM = 128
BATCH = 16384
NC_ROWS = 512
NUM_CORES = 2
NUM_SUBCORES = 16
NW = NUM_CORES * NUM_SUBCORES
RPW = BATCH // NW
CHUNK = 64
NCHUNK = RPW // CHUNK
SEG = DIM // 16
CL_PER_W = NC_ROWS // NW


def _sc_body(idxk, i1, i2, i3, i4, node, rela, link, wepad,
             nc0, wc0, tx0, nc1, wc1, tx1,
             scores, pc0, pc1,
             ix0, ix1, ix2, ix3, ix4, buf, scores_v,
             crows0, crows1, we_v, cidx0, cidx1, cwe0, cwe1, ctx0, ctx1,
             pv0, pv1,
             semI, semC, semC2, semS, semP, semA, semB):
  wid = lax.axis_index("s") * NUM_CORES + lax.axis_index("c")
  base = wid * RPW
  cb = wid * CL_PER_W

  hI = [
      pltpu.async_copy(idxk.at[pl.ds(base, RPW)], ix0, semI),
      pltpu.async_copy(i1.at[pl.ds(base, RPW)], ix1, semI),
      pltpu.async_copy(i2.at[pl.ds(base, RPW)], ix2, semI),
      pltpu.async_copy(i3.at[pl.ds(base, RPW)], ix3, semI),
      pltpu.async_copy(i4.at[pl.ds(base, RPW)], ix4, semI),
  ]
  hC = [
      pltpu.async_copy(nc0.at[pl.ds(cb, CL_PER_W)], cidx0, semC),
      pltpu.async_copy(nc1.at[pl.ds(cb, CL_PER_W)], cidx1, semC),
      pltpu.async_copy(wc0.at[pl.ds(cb, CL_PER_W)], cwe0, semC),
      pltpu.async_copy(wc1.at[pl.ds(cb, CL_PER_W)], cwe1, semC),
      pltpu.async_copy(tx0.at[pl.ds(cb, CL_PER_W)], ctx0, semC),
      pltpu.async_copy(tx1.at[pl.ds(cb, CL_PER_W)], ctx1, semC),
      pltpu.async_copy(wepad, we_v, semC),
  ]

  def fire(c, s, sem):
    o = c * CHUNK
    return [
        pltpu.async_copy(node.at[ix1.at[pl.ds(o, CHUNK)]], buf.at[s, 0], sem),
        pltpu.async_copy(node.at[ix2.at[pl.ds(o, CHUNK)]], buf.at[s, 1], sem),
        pltpu.async_copy(node.at[ix3.at[pl.ds(o, CHUNK)]], buf.at[s, 2], sem),
        pltpu.async_copy(node.at[ix4.at[pl.ds(o, CHUNK)]], buf.at[s, 3], sem),
        pltpu.async_copy(rela.at[ix0.at[pl.ds(o, CHUNK)]], buf.at[s, 4], sem),
        pltpu.async_copy(link.at[ix0.at[pl.ds(o, CHUNK)]], buf.at[s, 5], sem),
    ]

  def compute(s):
    def row_body(r, carry):
      acc = jnp.zeros((16,), jnp.float32)
      for g in range(SEG):
        d = pl.ds(g * 16, 16)
        pi = buf[s, 0, r, d]
        pj = buf[s, 1, r, d]
        ni = buf[s, 2, r, d]
        nj = buf[s, 3, r, d]
        rk = buf[s, 4, r, d]
        lk = buf[s, 5, r, d]
        acc = acc + (nj - ni + pi - pj) * rk + (ni * nj - pi * pj) * lk
      scores_v[s, r, :] = acc
      return carry
    lax.fori_loop(0, CHUNK, row_body, 0)

  def cluster(cwe_r, ctx_r, crows_r, pv_r, pc_out):
    we_vec = we_v[...]
    cwe_vec = cwe_r[...]
    wex = jnp.zeros((16,), jnp.float32)
    for t in range(8):
      wex = jnp.where(cwe_vec == t, we_vec[t], wex)
    coeff_vec = ctx_r[...] * wex
    crs = [coeff_vec[r] for r in range(CL_PER_W)]
    for g in range(SEG):
      d = pl.ds(g * 16, 16)
      acc = jnp.zeros((16,), jnp.float32)
      for r in range(CL_PER_W):
        acc = acc + crs[r] * crows_r[r, d]
      pv_r[d] = acc
    return pltpu.async_copy(pv_r, pc_out.at[wid], semP)

  for h in hC:
    h.wait()
  hG = [
      pltpu.async_copy(node.at[cidx0], crows0, semC2),
      pltpu.async_copy(node.at[cidx1], crows1, semC2),
  ]
  for h in hI:
    h.wait()
  sems = (semA, semB)
  pending = fire(0, 0, sems[0])
  for h in hG:
    h.wait()
  hP = [cluster(cwe0, ctx0, crows0, pv0, pc0),
        cluster(cwe1, ctx1, crows1, pv1, pc1)]

  hS = [None, None]
  for c in range(NCHUNK):
    s = c % 2
    nxt = None
    if c + 1 < NCHUNK:
      nxt = fire(c + 1, 1 - s, sems[1 - s])
    for h in pending:
      h.wait()
    if hS[s] is not None:
      hS[s].wait()
    compute(s)
    hS[s] = pltpu.async_copy(
        scores_v.at[s], scores.at[pl.ds(base + c * CHUNK, CHUNK)], semS)
    pending = nxt
  for h in hS:
    if h is not None:
      h.wait()
  for h in hP:
    h.wait()


@functools.partial(
    pl.kernel,
    out_type=(
        jax.ShapeDtypeStruct((BATCH, 16), jnp.float32),
        jax.ShapeDtypeStruct((NW, DIM), jnp.float32),
        jax.ShapeDtypeStruct((NW, DIM), jnp.float32),
    ),
    mesh=plsc.VectorSubcoreMesh(core_axis_name="c", subcore_axis_name="s"),
    scratch_types=[
        pltpu.VMEM((RPW,), jnp.int32),
        pltpu.VMEM((RPW,), jnp.int32),
        pltpu.VMEM((RPW,), jnp.int32),
        pltpu.VMEM((RPW,), jnp.int32),
        pltpu.VMEM((RPW,), jnp.int32),
        pltpu.VMEM((2, 6, CHUNK, DIM), jnp.float32),
        pltpu.VMEM((2, CHUNK, 16), jnp.float32),
        pltpu.VMEM((CL_PER_W, DIM), jnp.float32),
        pltpu.VMEM((CL_PER_W, DIM), jnp.float32),
        pltpu.VMEM((16,), jnp.float32),
        pltpu.VMEM((16,), jnp.int32),
        pltpu.VMEM((16,), jnp.int32),
        pltpu.VMEM((16,), jnp.int32),
        pltpu.VMEM((16,), jnp.int32),
        pltpu.VMEM((16,), jnp.float32),
        pltpu.VMEM((16,), jnp.float32),
        pltpu.VMEM((DIM,), jnp.float32),
        pltpu.VMEM((DIM,), jnp.float32),
        pltpu.SemaphoreType.DMA,
        pltpu.SemaphoreType.DMA,
        pltpu.SemaphoreType.DMA,
        pltpu.SemaphoreType.DMA,
        pltpu.SemaphoreType.DMA,
        pltpu.SemaphoreType.DMA,
        pltpu.SemaphoreType.DMA,
    ],
)
def _sc_kernel(*refs):
  _sc_body(*refs)


def _tc_body(scores_ref, pc0_ref, pc1_ref, out_ref):
  s = scores_ref[...]
  err = jnp.sum(s, axis=1, keepdims=True)
  sp = jnp.maximum(err, 0.0) + jnp.log1p(jnp.exp(-jnp.abs(err)))
  loss = jnp.sum(sp)
  c0 = jnp.sum(pc0_ref[...], axis=0) * (1.0 / (NC_ROWS * 1000.0))
  c1 = jnp.sum(pc1_ref[...], axis=0) * (1.0 / (NC_ROWS * 1500.0))
  dvec = c0 - c1
  dis = 0.5 * jnp.sqrt(jnp.sum(dvec * dvec))
  out_ref[0, 0] = loss - 2.0 * dis


_tc_kernel = pl.pallas_call(
    _tc_body,
    out_shape=jax.ShapeDtypeStruct((1, 1), jnp.float32),
    out_specs=pl.BlockSpec(memory_space=pltpu.SMEM),
)


def kernel(batchVector, nodeEmbedding, relaEmbedding, linkEmbedding, weEmbedding,
           node_c0, we_c0, timesx_c0, node_c1, we_c1, timesx_c1):
  cols = batchVector.T
  wepad = jnp.concatenate(
      [weEmbedding.reshape(-1), jnp.zeros((16 - weEmbedding.shape[0],), jnp.float32)])
  scores, pc0, pc1 = _sc_kernel(
      cols[0], cols[1], cols[2], cols[3], cols[4],
      nodeEmbedding, relaEmbedding, linkEmbedding, wepad,
      node_c0, we_c0, timesx_c0.reshape(-1),
      node_c1, we_c1, timesx_c1.reshape(-1))
  total = _tc_kernel(scores, pc0, pc1)
  return total[0, 0]


# speedup vs baseline: 1.9325x; 1.0191x over previous
import functools

import jax
import jax.numpy as jnp
from jax import lax
from jax.experimental import pallas as pl
from jax.experimental.pallas import tpu as pltpu
from jax.experimental.pallas import tpu_sc as plsc

DIM = 128
BATCH = 16384
NC_ROWS = 512
NUM_CORES = 2
NUM_SUBCORES = 16
NW = NUM_CORES * NUM_SUBCORES
RPW = BATCH // NW
CHUNK = 64
NCHUNK = RPW // CHUNK
SEG = DIM // 16
CL_PER_W = NC_ROWS // NW


def _sc_body(idxk, i1, i2, i3, i4, node, rela, link, we8,
             nc0, wc0, tx0, nc1, wc1, tx1,
             scores, pc0, pc1,
             ix0, ix1, ix2, ix3, ix4, buf, scores_v,
             crows0, crows1, we_v, cidx0, cidx1, cwe0, cwe1, ctx0, ctx1,
             pv0, pv1,
             semI, semC, semC2, semS, semP, semA, semB):
  wid = lax.axis_index("s") * NUM_CORES + lax.axis_index("c")
  base = wid * RPW
  cb = wid * CL_PER_W

  hI = [
      pltpu.async_copy(idxk.at[pl.ds(base, RPW)], ix0, semI),
      pltpu.async_copy(i1.at[pl.ds(base, RPW)], ix1, semI),
      pltpu.async_copy(i2.at[pl.ds(base, RPW)], ix2, semI),
      pltpu.async_copy(i3.at[pl.ds(base, RPW)], ix3, semI),
      pltpu.async_copy(i4.at[pl.ds(base, RPW)], ix4, semI),
  ]
  hC = [
      pltpu.async_copy(nc0.at[pl.ds(cb, CL_PER_W)], cidx0, semC),
      pltpu.async_copy(nc1.at[pl.ds(cb, CL_PER_W)], cidx1, semC),
      pltpu.async_copy(wc0.at[pl.ds(cb, CL_PER_W)], cwe0, semC),
      pltpu.async_copy(wc1.at[pl.ds(cb, CL_PER_W)], cwe1, semC),
      pltpu.async_copy(tx0.at[pl.ds(cb, CL_PER_W)], ctx0, semC),
      pltpu.async_copy(tx1.at[pl.ds(cb, CL_PER_W)], ctx1, semC),
      pltpu.async_copy(we8, we_v.at[pl.ds(0, 8)], semC),
  ]

  def fire(c, s, sem):
    o = c * CHUNK
    return [
        pltpu.async_copy(node.at[ix1.at[pl.ds(o, CHUNK)]], buf.at[s, 0], sem),
        pltpu.async_copy(node.at[ix2.at[pl.ds(o, CHUNK)]], buf.at[s, 1], sem),
        pltpu.async_copy(node.at[ix3.at[pl.ds(o, CHUNK)]], buf.at[s, 2], sem),
        pltpu.async_copy(node.at[ix4.at[pl.ds(o, CHUNK)]], buf.at[s, 3], sem),
        pltpu.async_copy(rela.at[ix0.at[pl.ds(o, CHUNK)]], buf.at[s, 4], sem),
        pltpu.async_copy(link.at[ix0.at[pl.ds(o, CHUNK)]], buf.at[s, 5], sem),
    ]

  def compute(c, s):
    def row_body(r, carry):
      acc = jnp.zeros((16,), jnp.float32)
      for g in range(SEG):
        d = pl.ds(g * 16, 16)
        pi = buf[s, 0, r, d]
        pj = buf[s, 1, r, d]
        ni = buf[s, 2, r, d]
        nj = buf[s, 3, r, d]
        rk = buf[s, 4, r, d]
        lk = buf[s, 5, r, d]
        acc = acc + (nj - ni + pi - pj) * rk + (ni * nj - pi * pj) * lk
      scores_v[s, r, :] = acc
      return carry
    lax.fori_loop(0, CHUNK, row_body, 0)

  def cluster(cwe_r, ctx_r, crows_r, pv_r, pc_out):
    we_vec = we_v[...]
    cwe_vec = cwe_r[...]
    wex = jnp.zeros((16,), jnp.float32)
    for t in range(8):
      wex = jnp.where(cwe_vec == t, we_vec[t], wex)
    coeff_vec = ctx_r[...] * wex
    crs = [coeff_vec[r] for r in range(CL_PER_W)]
    for g in range(SEG):
      d = pl.ds(g * 16, 16)
      acc = jnp.zeros((16,), jnp.float32)
      for r in range(CL_PER_W):
        acc = acc + crs[r] * crows_r[r, d]
      pv_r[d] = acc
    return pltpu.async_copy(pv_r, pc_out.at[wid], semP)

  for h in hC:
    h.wait()
  hG = [
      pltpu.async_copy(node.at[cidx0], crows0, semC2),
      pltpu.async_copy(node.at[cidx1], crows1, semC2),
  ]
  for h in hI:
    h.wait()
  sems = (semA, semB)
  pending = fire(0, 0, sems[0])
  for h in hG:
    h.wait()
  hP = [cluster(cwe0, ctx0, crows0, pv0, pc0),
        cluster(cwe1, ctx1, crows1, pv1, pc1)]

  hS = [None, None]
  for c in range(NCHUNK):
    s = c % 2
    nxt = None
    if c + 1 < NCHUNK:
      nxt = fire(c + 1, 1 - s, sems[1 - s])
    for h in pending:
      h.wait()
    if hS[s] is not None:
      hS[s].wait()
    compute(c, s)
    hS[s] = pltpu.async_copy(
        scores_v.at[s], scores.at[pl.ds(base + c * CHUNK, CHUNK)], semS)
    pending = nxt
  for h in hS:
    if h is not None:
      h.wait()
  for h in hP:
    h.wait()


@functools.partial(
    pl.kernel,
    out_type=(
        jax.ShapeDtypeStruct((BATCH, 16), jnp.float32),
        jax.ShapeDtypeStruct((NW, DIM), jnp.float32),
        jax.ShapeDtypeStruct((NW, DIM), jnp.float32),
    ),
    mesh=plsc.VectorSubcoreMesh(core_axis_name="c", subcore_axis_name="s"),
    scratch_types=[
        pltpu.VMEM((RPW,), jnp.int32),
        pltpu.VMEM((RPW,), jnp.int32),
        pltpu.VMEM((RPW,), jnp.int32),
        pltpu.VMEM((RPW,), jnp.int32),
        pltpu.VMEM((RPW,), jnp.int32),
        pltpu.VMEM((2, 6, CHUNK, DIM), jnp.float32),
        pltpu.VMEM((2, CHUNK, 16), jnp.float32),
        pltpu.VMEM((CL_PER_W, DIM), jnp.float32),
        pltpu.VMEM((CL_PER_W, DIM), jnp.float32),
        pltpu.VMEM((16,), jnp.float32),
        pltpu.VMEM((16,), jnp.int32),
        pltpu.VMEM((16,), jnp.int32),
        pltpu.VMEM((16,), jnp.int32),
        pltpu.VMEM((16,), jnp.int32),
        pltpu.VMEM((16,), jnp.float32),
        pltpu.VMEM((16,), jnp.float32),
        pltpu.VMEM((DIM,), jnp.float32),
        pltpu.VMEM((DIM,), jnp.float32),
        pltpu.SemaphoreType.DMA,
        pltpu.SemaphoreType.DMA,
        pltpu.SemaphoreType.DMA,
        pltpu.SemaphoreType.DMA,
        pltpu.SemaphoreType.DMA,
        pltpu.SemaphoreType.DMA,
        pltpu.SemaphoreType.DMA,
    ],
)
def _sc_kernel(*refs):
  _sc_body(*refs)


def _tc_body(scores_ref, pc0_ref, pc1_ref, out_ref):
  s = scores_ref[...]
  err = jnp.sum(s, axis=1, keepdims=True)
  sp = jnp.maximum(err, 0.0) + jnp.log1p(jnp.exp(-jnp.abs(err)))
  loss = jnp.sum(sp)
  c0 = jnp.sum(pc0_ref[...], axis=0) * (1.0 / (NC_ROWS * 1000.0))
  c1 = jnp.sum(pc1_ref[...], axis=0) * (1.0 / (NC_ROWS * 1500.0))
  dvec = c0 - c1
  dis = 0.5 * jnp.sqrt(jnp.sum(dvec * dvec))
  out_ref[0, 0] = loss - 2.0 * dis


_tc_kernel = pl.pallas_call(
    _tc_body,
    out_shape=jax.ShapeDtypeStruct((1, 1), jnp.float32),
    out_specs=pl.BlockSpec(memory_space=pltpu.SMEM),
)


def kernel(batchVector, nodeEmbedding, relaEmbedding, linkEmbedding, weEmbedding,
           node_c0, we_c0, timesx_c0, node_c1, we_c1, timesx_c1):
  cols = batchVector.T
  scores, pc0, pc1 = _sc_kernel(
      cols[0], cols[1], cols[2], cols[3], cols[4],
      nodeEmbedding, relaEmbedding, linkEmbedding, weEmbedding.reshape(-1),
      node_c0, we_c0, timesx_c0.reshape(-1),
      node_c1, we_c1, timesx_c1.reshape(-1))
  total = _tc_kernel(scores, pc0, pc1)
  return total[0, 0]
